# baseline (speedup 1.0000x reference)
import functools

import jax
import jax.numpy as jnp
import numpy as np
from jax import lax
from jax.experimental import pallas as pl
from jax.experimental.pallas import tpu as pltpu
from jax.experimental.pallas import tpu_sc as plsc

N_HEAD, D_K, D_V, D_MODEL = 4, 32, 32, 128
B_, E_, NBR_ = 1024, 16, 100
T_ = 2 * E_

BB1 = 16
BB2 = 16
BBT = BB2 * T_

NW = 32
TOK_PER_W = B_ * T_ // NW
TOK_CHUNK = 128
SEG_PER_W = (B_ * 2) // NW
NBR_PER_W = SEG_PER_W * NBR_
NPAIR = SEG_PER_W // 2

_INTERPRET = False


def _sc_body(mem_hbm, tok_hbm, nbr_hbm, xn_hbm, nsum_hbm,
             tokidx_a, tokidx_b, tokrows_a, tokrows_b,
             nbridx_v, rows_a, rows_b, rows_c, segsum_v,
             sem_ta, sem_tb, sem_a, sem_b, sem_c):
    wid = lax.axis_index("s") * 2 + lax.axis_index("c")

    def start_pair(pidx, buf, sem):
        off = pl.multiple_of(pidx * 2 * NBR_, 8)
        pltpu.async_copy(mem_hbm.at[nbridx_v.at[pl.ds(off, 128)]],
                         buf.at[pl.ds(0, 128)], sem)
        pltpu.async_copy(mem_hbm.at[nbridx_v.at[pl.ds(off + 128, 72)]],
                         buf.at[pl.ds(128, 72)], sem)

    def wait_pair(buf, sem):
        pltpu.make_async_copy(mem_hbm.at[nbridx_v.at[pl.ds(0, 128)]],
                              buf.at[pl.ds(0, 128)], sem).wait()
        pltpu.make_async_copy(mem_hbm.at[nbridx_v.at[pl.ds(0, 72)]],
                              buf.at[pl.ds(128, 72)], sem).wait()

    def accum_from(buf, seg0):
        def seg_sum(row0, seg):
            def body(jj, acc):
                r = row0 + 2 * jj
                acc = tuple(acc[c] + buf[r, pl.ds(16 * c, 16)]
                            for c in range(8))
                return tuple(acc[c] + buf[r + 1, pl.ds(16 * c, 16)]
                             for c in range(8))

            acc = lax.fori_loop(
                0, NBR_ // 2, body,
                tuple(jnp.zeros((16,), jnp.float32) for _ in range(8)))
            for c in range(8):
                segsum_v[seg, pl.ds(16 * c, 16)] = acc[c]

        seg_sum(0, seg0)
        seg_sum(NBR_, seg0 + 1)

    nbr_base = pl.multiple_of(wid * NBR_PER_W, 8)
    pltpu.sync_copy(nbr_hbm.at[pl.ds(nbr_base, NBR_PER_W)], nbridx_v)
    start_pair(0, rows_a, sem_a)
    start_pair(1, rows_b, sem_b)

    tok_base = wid * TOK_PER_W
    n_tok = TOK_PER_W // TOK_CHUNK

    def start_tok(i, idxbuf, rowsbuf, sem):
        base = pl.multiple_of(tok_base + i * TOK_CHUNK, 8)
        pltpu.sync_copy(tok_hbm.at[pl.ds(base, TOK_CHUNK)], idxbuf)
        pltpu.async_copy(mem_hbm.at[idxbuf], rowsbuf, sem)

    def finish_tok(i, idxbuf, rowsbuf, sem):
        base = pl.multiple_of(tok_base + i * TOK_CHUNK, 8)
        pltpu.make_async_copy(mem_hbm.at[idxbuf], rowsbuf, sem).wait()
        pltpu.sync_copy(rowsbuf, xn_hbm.at[pl.ds(base, TOK_CHUNK)])

    start_tok(0, tokidx_a, tokrows_a, sem_ta)

    def tok_body(ii, carry):
        start_tok(2 * ii + 1, tokidx_b, tokrows_b, sem_tb)
        finish_tok(2 * ii, tokidx_a, tokrows_a, sem_ta)

        @pl.when(2 * ii + 2 < n_tok)
        def _():
            start_tok(2 * ii + 2, tokidx_a, tokrows_a, sem_ta)

        finish_tok(2 * ii + 1, tokidx_b, tokrows_b, sem_tb)
        return carry

    lax.fori_loop(0, n_tok // 2, tok_body, 0)

    def pair_body(i, carry):
        p0 = 3 * i
        start_pair(p0 + 2, rows_c, sem_c)
        wait_pair(rows_a, sem_a)
        accum_from(rows_a, 2 * p0)
        start_pair(p0 + 3, rows_a, sem_a)
        wait_pair(rows_b, sem_b)
        accum_from(rows_b, 2 * p0 + 2)
        start_pair(p0 + 4, rows_b, sem_b)
        wait_pair(rows_c, sem_c)
        accum_from(rows_c, 2 * p0 + 4)
        return carry

    lax.fori_loop(0, (NPAIR - 2) // 3, pair_body, 0)
    wait_pair(rows_a, sem_a)
    accum_from(rows_a, 2 * (NPAIR - 2))
    wait_pair(rows_b, sem_b)
    accum_from(rows_b, 2 * (NPAIR - 1))
    out_base = pl.multiple_of(wid * SEG_PER_W, 8)
    pltpu.sync_copy(segsum_v, nsum_hbm.at[pl.ds(out_base, SEG_PER_W)])


def _sc_gather(memory, tok_ids, nbr_ids):
    mesh = plsc.VectorSubcoreMesh(core_axis_name="c", subcore_axis_name="s")
    f = pl.kernel(
        _sc_body, mesh=mesh,
        out_type=[
            jax.ShapeDtypeStruct((B_ * T_, D_MODEL), jnp.float32),
            jax.ShapeDtypeStruct((B_ * 2, D_MODEL), jnp.float32),
        ],
        scratch_types=[
            pltpu.VMEM((TOK_CHUNK,), jnp.int32),
            pltpu.VMEM((TOK_CHUNK,), jnp.int32),
            pltpu.VMEM((TOK_CHUNK, D_MODEL), jnp.float32),
            pltpu.VMEM((TOK_CHUNK, D_MODEL), jnp.float32),
            pltpu.VMEM((NBR_PER_W,), jnp.int32),
            pltpu.VMEM((2 * NBR_, D_MODEL), jnp.float32),
            pltpu.VMEM((2 * NBR_, D_MODEL), jnp.float32),
            pltpu.VMEM((2 * NBR_, D_MODEL), jnp.float32),
            pltpu.VMEM((SEG_PER_W, D_MODEL), jnp.float32),
            pltpu.SemaphoreType.DMA,
            pltpu.SemaphoreType.DMA,
            pltpu.SemaphoreType.DMA,
            pltpu.SemaphoreType.DMA,
            pltpu.SemaphoreType.DMA,
        ],
    )
    return f(memory, tok_ids, nbr_ids)


_CC = (0.9999992107411736, -0.4999942131496052, 0.04165977758570175,
       -0.001385878920444182, 2.4202932052956594e-05, -2.1972921876445284e-07)
_INV_2PI = 0.15915494309189535
_TWO_PI = 6.283185307179586


def _fast_cos(x):
    k = jnp.floor(x * _INV_2PI + 0.5)
    r = x - k * _TWO_PI
    t = r * r
    p = _CC[5]
    for c in (_CC[4], _CC[3], _CC[2], _CC[1], _CC[0]):
        p = p * t + c
    return p


FBLK_D = 2 * BB2 * NBR_
NB_D = B_ // BB2


def _mm(a, b):
    return lax.dot_general(a, b, (((1,), (0,)), ((), ())),
                           preferred_element_type=jnp.float32)


def _mm_nt(a, b):
    return lax.dot_general(a, b, (((1,), (1,)), ((), ())),
                           preferred_element_type=jnp.float32)


def _tfeat_block(dtr, w_col, b_col, segT):
    ang = dtr * w_col + b_col
    c = _fast_cos(ang).astype(jnp.bfloat16)
    return lax.dot_general(segT, c, (((1,), (1,)), ((), ())),
                           preferred_element_type=jnp.float32)


def _dense_body(xn_ref, nsum_ref, dtr0_ref, dtrn_ref, wt_ref, bt_ref,
                segt_ref, idc_ref, bd_ref,
                wenc_ref, benc_ref, wq_ref, wo_ref,
                wc_ref, bc_ref, x_ref, emb_ref, lb_ref, tsum_scr):
    f32 = jnp.float32
    bf16 = jnp.bfloat16

    @pl.when(pl.program_id(0) == 0)
    def _():
        tsum_scr[...] = _tfeat_block(dtr0_ref[...], wt_ref[...],
                                     bt_ref[...], segt_ref[...])

    xn = xn_ref[...]
    agg = (nsum_ref[...] + tsum_scr[...]) * (1.0 / (NBR_ + 1e-9))

    aggx = jnp.broadcast_to(agg[:, None, :], (2 * BB2, E_, D_MODEL))
    x_in = xn + aggx.reshape(BBT, D_MODEL)
    xe = jnp.tanh(_mm(x_in.astype(bf16), wenc_ref[...])
                  + benc_ref[...])
    xr = xe.reshape(BB2, 2, E_, D_MODEL)
    x_ref[0] = xr[:, 0]
    x_ref[1] = xr[:, 1]

    mcol = (idc_ref[...] != 0).astype(f32)
    mcolb = (idc_ref[...] != 0).astype(bf16)
    bd = bd_ref[...]

    xeb = xe.astype(bf16)
    qkv = _mm(xeb, wq_ref[...]).astype(bf16)
    q = lax.slice(qkv, (0, 0), (BBT, D_MODEL))
    k = lax.slice(qkv, (0, D_MODEL), (BBT, 2 * D_MODEL))
    v = lax.slice(qkv, (0, 2 * D_MODEL), (BBT, 3 * D_MODEL))
    outacc = jnp.zeros((BBT, D_MODEL), f32)
    for h in range(N_HEAD):
        qh = lax.slice(q, (0, h * D_K), (BBT, (h + 1) * D_K))
        kh = lax.slice(k, (0, h * D_K), (BBT, (h + 1) * D_K))
        vh = lax.slice(v, (0, h * D_V), (BBT, (h + 1) * D_V))
        sh = _mm_nt(qh, kh).astype(bf16)
        p = jnp.exp(jnp.minimum(sh, bf16(80.0))) * bd * mcolb
        dn = jnp.sum(p, axis=1, keepdims=True).astype(f32) + 1e-30
        oh = _mm(p, vh) / dn
        outacc = outacc + _mm(oh.astype(bf16), wo_ref[h])

    ri2 = lax.broadcasted_iota(jnp.int32, (BB2, BBT), 0)
    ci2 = lax.broadcasted_iota(jnp.int32, (BB2, BBT), 1)
    sel = ((ci2 // T_) == ri2).astype(f32) * mcol
    cnt = jnp.sum(sel, axis=1, keepdims=True)
    emb = _mm(sel, outacc) / (cnt + 1e-9)
    emb_ref[...] = emb
    lb_ref[...] = jax.nn.sigmoid(_mm(emb, wc_ref[...]) + bc_ref[...])

    tsum_scr[...] = _tfeat_block(dtrn_ref[...], wt_ref[...],
                                 bt_ref[...], segt_ref[...])


def _dense(xn, nsum, dtrow, wt, bt, segt, idc, bd, wenc, benc, wqkv, wo4,
           wc, bc2):
    return pl.pallas_call(
        _dense_body,
        grid=(NB_D,),
        in_specs=[
            pl.BlockSpec((BBT, D_MODEL), lambda i: (i, 0)),
            pl.BlockSpec((2 * BB2, D_MODEL), lambda i: (i, 0)),
            pl.BlockSpec((1, FBLK_D), lambda i: (0, 0)),
            pl.BlockSpec((1, FBLK_D),
                         lambda i: (0, jnp.minimum(i + 1, NB_D - 1))),
            pl.BlockSpec((D_MODEL, 1), lambda i: (0, 0)),
            pl.BlockSpec((D_MODEL, 1), lambda i: (0, 0)),
            pl.BlockSpec((2 * BB2, FBLK_D), lambda i: (0, 0)),
            pl.BlockSpec((1, BBT), lambda i: (0, i)),
            pl.BlockSpec((BBT, BBT), lambda i: (0, 0)),
            pl.BlockSpec((D_MODEL, D_MODEL), lambda i: (0, 0)),
            pl.BlockSpec((1, D_MODEL), lambda i: (0, 0)),
            pl.BlockSpec((D_MODEL, 3 * D_MODEL), lambda i: (0, 0)),
            pl.BlockSpec((N_HEAD, D_V, D_MODEL), lambda i: (0, 0, 0)),
            pl.BlockSpec((D_MODEL, 1), lambda i: (0, 0)),
            pl.BlockSpec((1, 1), lambda i: (0, 0)),
        ],
        out_specs=[
            pl.BlockSpec((2, BB2, E_, D_MODEL), lambda i: (0, i, 0, 0)),
            pl.BlockSpec((BB2, D_MODEL), lambda i: (i, 0)),
            pl.BlockSpec((BB2, 1), lambda i: (i, 0)),
        ],
        out_shape=[
            jax.ShapeDtypeStruct((2, B_, E_, D_MODEL), jnp.float32),
            jax.ShapeDtypeStruct((B_, D_MODEL), jnp.float32),
            jax.ShapeDtypeStruct((B_, 1), jnp.float32),
        ],
        interpret=_INTERPRET,
        scratch_shapes=[pltpu.VMEM((2 * BB2, D_MODEL), jnp.float32)],
    )(xn, nsum, dtrow, dtrow, wt, bt, segt, idc, bd, wenc, benc, wqkv, wo4,
      wc, bc2)


def kernel(memory, batch_hyperedge, batch_h_index, cur_time,
           batch_h_index_times, batch_h_index_mask, W_enc, b_enc, w_time,
           b_time, Wq, Wk, Wv, Wo, Wc, bc):
    bh = batch_hyperedge.astype(jnp.int32)
    ids_tok = jnp.concatenate([bh[0], bh[1]], axis=1)
    ids_flat = ids_tok.reshape(B_ * T_)
    nbr_ids = batch_h_index.astype(jnp.int32).reshape(B_ * 2 * NBR_)

    xn, nsum = _sc_gather(memory, ids_flat, nbr_ids)

    dtrow = (cur_time[:, :, None] - batch_h_index_times).reshape(1, B_ * 2 * NBR_)

    bf16 = jnp.bfloat16
    segt = jnp.asarray(
        np.kron(np.eye(2 * BB2, dtype=np.float32),
                np.ones((1, NBR_), np.float32)),
        dtype=bf16)
    wqkv = jnp.concatenate(
        [Wq * (1.0 / np.sqrt(D_K)), Wk, Wv], axis=1).astype(bf16)
    wo4 = Wo.reshape(N_HEAD, D_V, D_MODEL).astype(bf16)

    bd = jnp.asarray(
        np.kron(np.eye(BB2, dtype=np.float32),
                np.ones((T_, T_), np.float32)), dtype=bf16)
    x4, emb, lb = _dense(
        xn, nsum, dtrow, w_time.reshape(D_MODEL, 1), b_time.reshape(D_MODEL, 1),
        segt, ids_flat.reshape(1, B_ * T_), bd,
        W_enc.astype(bf16), b_enc.reshape(1, D_MODEL), wqkv, wo4,
        Wc, bc.reshape(1, 1))
    return lb, emb, x4


# speedup vs baseline: 1.0613x; 1.0613x over previous
import functools

import jax
import jax.numpy as jnp
import numpy as np
from jax import lax
from jax.experimental import pallas as pl
from jax.experimental.pallas import tpu as pltpu
from jax.experimental.pallas import tpu_sc as plsc

N_HEAD, D_K, D_V, D_MODEL = 4, 32, 32, 128
B_, E_, NBR_ = 1024, 16, 100
T_ = 2 * E_

BB1 = 16
BB2 = 16
BBT = BB2 * T_

NW = 32
TOK_PER_W = B_ * T_ // NW
TOK_CHUNK = 128
SEG_PER_W = (B_ * 2) // NW
NBR_PER_W = SEG_PER_W * NBR_
NPAIR = SEG_PER_W // 2

_INTERPRET = False


def _sc_body(mem_hbm, tok_hbm, nbr_hbm, xn_hbm, nsum_hbm,
             tokidx_a, tokidx_b, tokrows_a, tokrows_b,
             nbridx_v, rows_a, rows_b, rows_c, segsum_v,
             sem_ta, sem_tb, sem_a, sem_b, sem_c):
    wid = lax.axis_index("s") * 2 + lax.axis_index("c")

    def start_pair(pidx, buf, sem):
        off = pl.multiple_of(pidx * 2 * NBR_, 8)
        pltpu.async_copy(mem_hbm.at[nbridx_v.at[pl.ds(off, 128)]],
                         buf.at[pl.ds(0, 128)], sem)
        pltpu.async_copy(mem_hbm.at[nbridx_v.at[pl.ds(off + 128, 72)]],
                         buf.at[pl.ds(128, 72)], sem)

    def wait_pair(buf, sem):
        pltpu.make_async_copy(mem_hbm.at[nbridx_v.at[pl.ds(0, 128)]],
                              buf.at[pl.ds(0, 128)], sem).wait()
        pltpu.make_async_copy(mem_hbm.at[nbridx_v.at[pl.ds(0, 72)]],
                              buf.at[pl.ds(128, 72)], sem).wait()

    def accum_from(buf, seg0):
        def seg_sum(row0, seg):
            def body(jj, acc):
                r = row0 + 2 * jj
                acc = tuple(acc[c] + buf[r, pl.ds(16 * c, 16)]
                            for c in range(8))
                return tuple(acc[c] + buf[r + 1, pl.ds(16 * c, 16)]
                             for c in range(8))

            acc = lax.fori_loop(
                0, NBR_ // 2, body,
                tuple(jnp.zeros((16,), jnp.float32) for _ in range(8)))
            for c in range(8):
                segsum_v[seg, pl.ds(16 * c, 16)] = acc[c]

        seg_sum(0, seg0)
        seg_sum(NBR_, seg0 + 1)

    nbr_base = pl.multiple_of(wid * NBR_PER_W, 8)
    pltpu.sync_copy(nbr_hbm.at[pl.ds(nbr_base, NBR_PER_W)], nbridx_v)
    start_pair(0, rows_a, sem_a)
    start_pair(1, rows_b, sem_b)

    tok_base = wid * TOK_PER_W
    n_tok = TOK_PER_W // TOK_CHUNK

    def start_tok(i, idxbuf, rowsbuf, sem):
        base = pl.multiple_of(tok_base + i * TOK_CHUNK, 8)
        pltpu.sync_copy(tok_hbm.at[pl.ds(base, TOK_CHUNK)], idxbuf)
        pltpu.async_copy(mem_hbm.at[idxbuf], rowsbuf, sem)

    def finish_tok(i, idxbuf, rowsbuf, sem):
        base = pl.multiple_of(tok_base + i * TOK_CHUNK, 8)
        pltpu.make_async_copy(mem_hbm.at[idxbuf], rowsbuf, sem).wait()
        pltpu.sync_copy(rowsbuf, xn_hbm.at[pl.ds(base, TOK_CHUNK)])

    start_tok(0, tokidx_a, tokrows_a, sem_ta)

    def tok_body(ii, carry):
        start_tok(2 * ii + 1, tokidx_b, tokrows_b, sem_tb)
        finish_tok(2 * ii, tokidx_a, tokrows_a, sem_ta)

        @pl.when(2 * ii + 2 < n_tok)
        def _():
            start_tok(2 * ii + 2, tokidx_a, tokrows_a, sem_ta)

        finish_tok(2 * ii + 1, tokidx_b, tokrows_b, sem_tb)
        return carry

    lax.fori_loop(0, n_tok // 2, tok_body, 0)

    def pair_body(i, carry):
        p0 = 3 * i
        start_pair(p0 + 2, rows_c, sem_c)
        wait_pair(rows_a, sem_a)
        accum_from(rows_a, 2 * p0)
        start_pair(p0 + 3, rows_a, sem_a)
        wait_pair(rows_b, sem_b)
        accum_from(rows_b, 2 * p0 + 2)
        start_pair(p0 + 4, rows_b, sem_b)
        wait_pair(rows_c, sem_c)
        accum_from(rows_c, 2 * p0 + 4)
        return carry

    lax.fori_loop(0, (NPAIR - 2) // 3, pair_body, 0)
    wait_pair(rows_a, sem_a)
    accum_from(rows_a, 2 * (NPAIR - 2))
    wait_pair(rows_b, sem_b)
    accum_from(rows_b, 2 * (NPAIR - 1))
    out_base = pl.multiple_of(wid * SEG_PER_W, 8)
    pltpu.sync_copy(segsum_v, nsum_hbm.at[pl.ds(out_base, SEG_PER_W)])


def _sc_gather(memory, tok_ids, nbr_ids):
    mesh = plsc.VectorSubcoreMesh(core_axis_name="c", subcore_axis_name="s")
    f = pl.kernel(
        _sc_body, mesh=mesh,
        out_type=[
            jax.ShapeDtypeStruct((B_ * T_, D_MODEL), jnp.float32),
            jax.ShapeDtypeStruct((B_ * 2, D_MODEL), jnp.float32),
        ],
        scratch_types=[
            pltpu.VMEM((TOK_CHUNK,), jnp.int32),
            pltpu.VMEM((TOK_CHUNK,), jnp.int32),
            pltpu.VMEM((TOK_CHUNK, D_MODEL), jnp.float32),
            pltpu.VMEM((TOK_CHUNK, D_MODEL), jnp.float32),
            pltpu.VMEM((NBR_PER_W,), jnp.int32),
            pltpu.VMEM((2 * NBR_, D_MODEL), jnp.float32),
            pltpu.VMEM((2 * NBR_, D_MODEL), jnp.float32),
            pltpu.VMEM((2 * NBR_, D_MODEL), jnp.float32),
            pltpu.VMEM((SEG_PER_W, D_MODEL), jnp.float32),
            pltpu.SemaphoreType.DMA,
            pltpu.SemaphoreType.DMA,
            pltpu.SemaphoreType.DMA,
            pltpu.SemaphoreType.DMA,
            pltpu.SemaphoreType.DMA,
        ],
    )
    return f(memory, tok_ids, nbr_ids)


_CC = (0.9999992107411736, -0.4999942131496052, 0.04165977758570175,
       -0.001385878920444182, 2.4202932052956594e-05, -2.1972921876445284e-07)
_INV_2PI = 0.15915494309189535
_TWO_PI = 6.283185307179586


def _fast_cos(x):
    k = jnp.floor(x * _INV_2PI + 0.5)
    r = x - k * _TWO_PI
    t = r * r
    p = _CC[5]
    for c in (_CC[4], _CC[3], _CC[2], _CC[1], _CC[0]):
        p = p * t + c
    return p


SEGBLK = 128
FBLK = SEGBLK * NBR_


def _tfeat_body(dtr_ref, w_ref, b_ref, seg_ref, out_ref):
    ang = dtr_ref[...] * w_ref[...] + b_ref[...]
    c = _fast_cos(ang).astype(jnp.bfloat16)
    out_ref[...] = lax.dot_general(
        c, seg_ref[...], (((1,), (0,)), ((), ())),
        preferred_element_type=jnp.float32)


def _tfeat_sum(dtrow, w_col, b_col, seg_mat):
    return pl.pallas_call(
        _tfeat_body,
        grid=(B_ * 2 // SEGBLK,),
        in_specs=[
            pl.BlockSpec((1, FBLK), lambda i: (0, i)),
            pl.BlockSpec((D_MODEL, 1), lambda i: (0, 0)),
            pl.BlockSpec((D_MODEL, 1), lambda i: (0, 0)),
            pl.BlockSpec((FBLK, SEGBLK), lambda i: (0, 0)),
        ],
        out_specs=pl.BlockSpec((D_MODEL, SEGBLK), lambda i: (0, i)),
        out_shape=jax.ShapeDtypeStruct((D_MODEL, B_ * 2), jnp.float32),
        interpret=_INTERPRET,
    )(dtrow, w_col, b_col, seg_mat)


def _mm(a, b):
    return lax.dot_general(a, b, (((1,), (0,)), ((), ())),
                           preferred_element_type=jnp.float32)


def _mm_nt(a, b):
    return lax.dot_general(a, b, (((1,), (1,)), ((), ())),
                           preferred_element_type=jnp.float32)


def _dense_body(xn_ref, nsum_ref, tsum_ref, idc_ref, bd_ref,
                wenc_ref, benc_ref, wq_ref, wo_ref,
                wc_ref, bc_ref, x_ref, emb_ref, lb_ref):
    f32 = jnp.float32
    bf16 = jnp.bfloat16
    xn = xn_ref[...]
    agg = (nsum_ref[...] + tsum_ref[...]) * (1.0 / (NBR_ + 1e-9))

    aggx = jnp.broadcast_to(agg[:, None, :], (2 * BB2, E_, D_MODEL))
    x_in = xn + aggx.reshape(BBT, D_MODEL)
    xe = jnp.tanh(_mm(x_in.astype(bf16), wenc_ref[...])
                  + benc_ref[...])
    xr = xe.reshape(BB2, 2, E_, D_MODEL)
    x_ref[0] = xr[:, 0]
    x_ref[1] = xr[:, 1]

    mcol = (idc_ref[...] != 0).astype(f32)
    mcolb = (idc_ref[...] != 0).astype(bf16)
    bd = bd_ref[...]

    xeb = xe.astype(bf16)
    qkv = _mm(xeb, wq_ref[...]).astype(bf16)
    q = lax.slice(qkv, (0, 0), (BBT, D_MODEL))
    k = lax.slice(qkv, (0, D_MODEL), (BBT, 2 * D_MODEL))
    v = lax.slice(qkv, (0, 2 * D_MODEL), (BBT, 3 * D_MODEL))
    outacc = jnp.zeros((BBT, D_MODEL), f32)
    for h in range(N_HEAD):
        qh = lax.slice(q, (0, h * D_K), (BBT, (h + 1) * D_K))
        kh = lax.slice(k, (0, h * D_K), (BBT, (h + 1) * D_K))
        vh = lax.slice(v, (0, h * D_V), (BBT, (h + 1) * D_V))
        sh = _mm_nt(qh, kh).astype(bf16)
        p = jnp.exp(jnp.minimum(sh, bf16(80.0))) * bd * mcolb
        dn = jnp.sum(p, axis=1, keepdims=True).astype(f32) + 1e-30
        oh = _mm(p, vh) / dn
        outacc = outacc + _mm(oh.astype(bf16), wo_ref[h])

    ri2 = lax.broadcasted_iota(jnp.int32, (BB2, BBT), 0)
    ci2 = lax.broadcasted_iota(jnp.int32, (BB2, BBT), 1)
    sel = ((ci2 // T_) == ri2).astype(f32) * mcol
    cnt = jnp.sum(sel, axis=1, keepdims=True)
    emb = _mm(sel, outacc) / (cnt + 1e-9)
    emb_ref[...] = emb
    lb_ref[...] = jax.nn.sigmoid(_mm(emb, wc_ref[...]) + bc_ref[...])


def _dense(xn, nsum, tsum, idc, bd, wenc, benc, wqkv, wo4, wc, bc2):
    return pl.pallas_call(
        _dense_body,
        grid=(B_ // BB2,),
        in_specs=[
            pl.BlockSpec((BBT, D_MODEL), lambda i: (i, 0)),
            pl.BlockSpec((2 * BB2, D_MODEL), lambda i: (i, 0)),
            pl.BlockSpec((2 * BB2, D_MODEL), lambda i: (i, 0)),
            pl.BlockSpec((1, BBT), lambda i: (0, i)),
            pl.BlockSpec((BBT, BBT), lambda i: (0, 0)),
            pl.BlockSpec((D_MODEL, D_MODEL), lambda i: (0, 0)),
            pl.BlockSpec((1, D_MODEL), lambda i: (0, 0)),
            pl.BlockSpec((D_MODEL, 3 * D_MODEL), lambda i: (0, 0)),
            pl.BlockSpec((N_HEAD, D_V, D_MODEL), lambda i: (0, 0, 0)),
            pl.BlockSpec((D_MODEL, 1), lambda i: (0, 0)),
            pl.BlockSpec((1, 1), lambda i: (0, 0)),
        ],
        out_specs=[
            pl.BlockSpec((2, BB2, E_, D_MODEL), lambda i: (0, i, 0, 0)),
            pl.BlockSpec((BB2, D_MODEL), lambda i: (i, 0)),
            pl.BlockSpec((BB2, 1), lambda i: (i, 0)),
        ],
        out_shape=[
            jax.ShapeDtypeStruct((2, B_, E_, D_MODEL), jnp.float32),
            jax.ShapeDtypeStruct((B_, D_MODEL), jnp.float32),
            jax.ShapeDtypeStruct((B_, 1), jnp.float32),
        ],
        interpret=_INTERPRET,
    )(xn, nsum, tsum, idc, bd, wenc, benc, wqkv, wo4, wc, bc2)


def kernel(memory, batch_hyperedge, batch_h_index, cur_time,
           batch_h_index_times, batch_h_index_mask, W_enc, b_enc, w_time,
           b_time, Wq, Wk, Wv, Wo, Wc, bc):
    bh = batch_hyperedge.astype(jnp.int32)
    ids_tok = jnp.concatenate([bh[0], bh[1]], axis=1)
    ids_flat = ids_tok.reshape(B_ * T_)
    nbr_ids = batch_h_index.astype(jnp.int32).reshape(B_ * 2 * NBR_)

    xn, nsum = _sc_gather(memory, ids_flat, nbr_ids)

    dtrow = (cur_time[:, :, None] - batch_h_index_times).reshape(1, B_ * 2 * NBR_)

    bf16 = jnp.bfloat16
    seg_mat = jnp.asarray(
        np.kron(np.eye(SEGBLK, dtype=np.float32),
                np.ones((NBR_, 1), np.float32)),
        dtype=bf16)
    tsum = _tfeat_sum(dtrow, w_time.reshape(D_MODEL, 1),
                      b_time.reshape(D_MODEL, 1), seg_mat).T

    wqkv = jnp.concatenate(
        [Wq * (1.0 / np.sqrt(D_K)), Wk, Wv], axis=1).astype(bf16)
    wo4 = Wo.reshape(N_HEAD, D_V, D_MODEL).astype(bf16)

    bd = jnp.asarray(
        np.kron(np.eye(BB2, dtype=np.float32),
                np.ones((T_, T_), np.float32)), dtype=bf16)
    x4, emb, lb = _dense(
        xn, nsum, tsum,
        ids_flat.reshape(1, B_ * T_), bd,
        W_enc.astype(bf16), b_enc.reshape(1, D_MODEL), wqkv, wo4,
        Wc, bc.reshape(1, 1))
    return lb, emb, x4


# speedup vs baseline: 1.2201x; 1.1496x over previous
import functools

import jax
import jax.numpy as jnp
import numpy as np
from jax import lax
from jax.experimental import pallas as pl
from jax.experimental.pallas import tpu as pltpu
from jax.experimental.pallas import tpu_sc as plsc

N_HEAD, D_K, D_V, D_MODEL = 4, 32, 32, 128
B_, E_, NBR_ = 1024, 16, 100
T_ = 2 * E_

BB1 = 16
BB2 = 16
BBT = BB2 * T_

NW = 32
TOK_PER_W = B_ * T_ // NW
TOK_CHUNK = 128
SEG_PER_W = (B_ * 2) // NW
NBR_PER_W = SEG_PER_W * NBR_
NPAIR = SEG_PER_W // 2

_INTERPRET = False


def _sc_body(mem_hbm, tok_hbm, nbr_hbm, xn_hbm, nsum_hbm,
             tokidx_a, tokidx_b, tokrows_a, tokrows_b,
             nbridx_v, rows_a, rows_b, rows_c, segsum_v,
             sem_ta, sem_tb, sem_a, sem_b, sem_c):
    wid = lax.axis_index("s") * 2 + lax.axis_index("c")

    def start_pair(pidx, buf, sem):
        off = pl.multiple_of(pidx * 2 * NBR_, 8)
        pltpu.async_copy(mem_hbm.at[nbridx_v.at[pl.ds(off, 128)]],
                         buf.at[pl.ds(0, 128)], sem)
        pltpu.async_copy(mem_hbm.at[nbridx_v.at[pl.ds(off + 128, 72)]],
                         buf.at[pl.ds(128, 72)], sem)

    def wait_pair(buf, sem):
        pltpu.make_async_copy(mem_hbm.at[nbridx_v.at[pl.ds(0, 128)]],
                              buf.at[pl.ds(0, 128)], sem).wait()
        pltpu.make_async_copy(mem_hbm.at[nbridx_v.at[pl.ds(0, 72)]],
                              buf.at[pl.ds(128, 72)], sem).wait()

    def accum_from(buf, seg0):
        def seg_sum(row0, seg):
            def body(jj, acc):
                r = row0 + 2 * jj
                acc = tuple(acc[c] + buf[r, pl.ds(16 * c, 16)]
                            for c in range(8))
                return tuple(acc[c] + buf[r + 1, pl.ds(16 * c, 16)]
                             for c in range(8))

            acc = lax.fori_loop(
                0, NBR_ // 2, body,
                tuple(jnp.zeros((16,), jnp.float32) for _ in range(8)))
            for c in range(8):
                segsum_v[seg, pl.ds(16 * c, 16)] = acc[c]

        seg_sum(0, seg0)
        seg_sum(NBR_, seg0 + 1)

    nbr_base = pl.multiple_of(wid * NBR_PER_W, 8)
    pltpu.sync_copy(nbr_hbm.at[pl.ds(nbr_base, NBR_PER_W)], nbridx_v)
    start_pair(0, rows_a, sem_a)
    start_pair(1, rows_b, sem_b)

    tok_base = wid * TOK_PER_W
    n_tok = TOK_PER_W // TOK_CHUNK

    def start_tok(i, idxbuf, rowsbuf, sem):
        base = pl.multiple_of(tok_base + i * TOK_CHUNK, 8)
        pltpu.sync_copy(tok_hbm.at[pl.ds(base, TOK_CHUNK)], idxbuf)
        pltpu.async_copy(mem_hbm.at[idxbuf], rowsbuf, sem)

    def finish_tok(i, idxbuf, rowsbuf, sem):
        base = pl.multiple_of(tok_base + i * TOK_CHUNK, 8)
        pltpu.make_async_copy(mem_hbm.at[idxbuf], rowsbuf, sem).wait()
        pltpu.sync_copy(rowsbuf, xn_hbm.at[pl.ds(base, TOK_CHUNK)])

    start_tok(0, tokidx_a, tokrows_a, sem_ta)

    def tok_body(ii, carry):
        start_tok(2 * ii + 1, tokidx_b, tokrows_b, sem_tb)
        finish_tok(2 * ii, tokidx_a, tokrows_a, sem_ta)

        @pl.when(2 * ii + 2 < n_tok)
        def _():
            start_tok(2 * ii + 2, tokidx_a, tokrows_a, sem_ta)

        finish_tok(2 * ii + 1, tokidx_b, tokrows_b, sem_tb)
        return carry

    lax.fori_loop(0, n_tok // 2, tok_body, 0)

    def pair_body(i, carry):
        p0 = 3 * i
        start_pair(p0 + 2, rows_c, sem_c)
        wait_pair(rows_a, sem_a)
        accum_from(rows_a, 2 * p0)
        start_pair(p0 + 3, rows_a, sem_a)
        wait_pair(rows_b, sem_b)
        accum_from(rows_b, 2 * p0 + 2)
        start_pair(p0 + 4, rows_b, sem_b)
        wait_pair(rows_c, sem_c)
        accum_from(rows_c, 2 * p0 + 4)
        return carry

    lax.fori_loop(0, (NPAIR - 2) // 3, pair_body, 0)
    wait_pair(rows_a, sem_a)
    accum_from(rows_a, 2 * (NPAIR - 2))
    wait_pair(rows_b, sem_b)
    accum_from(rows_b, 2 * (NPAIR - 1))
    out_base = pl.multiple_of(wid * SEG_PER_W, 8)
    pltpu.sync_copy(segsum_v, nsum_hbm.at[pl.ds(out_base, SEG_PER_W)])


def _sc_gather(memory, tok_ids, nbr_ids):
    mesh = plsc.VectorSubcoreMesh(core_axis_name="c", subcore_axis_name="s")
    f = pl.kernel(
        _sc_body, mesh=mesh,
        out_type=[
            jax.ShapeDtypeStruct((B_ * T_, D_MODEL), jnp.float32),
            jax.ShapeDtypeStruct((B_ * 2, D_MODEL), jnp.float32),
        ],
        scratch_types=[
            pltpu.VMEM((TOK_CHUNK,), jnp.int32),
            pltpu.VMEM((TOK_CHUNK,), jnp.int32),
            pltpu.VMEM((TOK_CHUNK, D_MODEL), jnp.float32),
            pltpu.VMEM((TOK_CHUNK, D_MODEL), jnp.float32),
            pltpu.VMEM((NBR_PER_W,), jnp.int32),
            pltpu.VMEM((2 * NBR_, D_MODEL), jnp.float32),
            pltpu.VMEM((2 * NBR_, D_MODEL), jnp.float32),
            pltpu.VMEM((2 * NBR_, D_MODEL), jnp.float32),
            pltpu.VMEM((SEG_PER_W, D_MODEL), jnp.float32),
            pltpu.SemaphoreType.DMA,
            pltpu.SemaphoreType.DMA,
            pltpu.SemaphoreType.DMA,
            pltpu.SemaphoreType.DMA,
            pltpu.SemaphoreType.DMA,
        ],
    )
    return f(memory, tok_ids, nbr_ids)


_CC = (0.999959018867681, -0.49979060076224385, 0.041494737249356074,
       -0.0013390575581256683, 1.8781276700241545e-05)
_INV_2PI = 0.15915494309189535
_TWO_PI = 6.283185307179586


def _fast_cos(x):
    k = jnp.floor(x * _INV_2PI + 0.5)
    r = x - k * _TWO_PI
    t = r * r
    p = _CC[4]
    for c in (_CC[3], _CC[2], _CC[1], _CC[0]):
        p = p * t + c
    return p


SEGBLK = 128
FBLK = SEGBLK * NBR_


def _tfeat_body(dtr_ref, w_ref, b_ref, seg_ref, out_ref):
    ang = dtr_ref[...] * w_ref[...] + b_ref[...]
    c = _fast_cos(ang).astype(jnp.bfloat16)
    out_ref[...] = lax.dot_general(
        c, seg_ref[...], (((1,), (0,)), ((), ())),
        preferred_element_type=jnp.float32)


def _tfeat_sum(dtrow, w_col, b_col, seg_mat):
    return pl.pallas_call(
        _tfeat_body,
        grid=(B_ * 2 // SEGBLK,),
        in_specs=[
            pl.BlockSpec((1, FBLK), lambda i: (0, i)),
            pl.BlockSpec((D_MODEL, 1), lambda i: (0, 0)),
            pl.BlockSpec((D_MODEL, 1), lambda i: (0, 0)),
            pl.BlockSpec((FBLK, SEGBLK), lambda i: (0, 0)),
        ],
        out_specs=pl.BlockSpec((D_MODEL, SEGBLK), lambda i: (0, i)),
        out_shape=jax.ShapeDtypeStruct((D_MODEL, B_ * 2), jnp.float32),
        interpret=_INTERPRET,
    )(dtrow, w_col, b_col, seg_mat)


def _mm(a, b):
    return lax.dot_general(a, b, (((1,), (0,)), ((), ())),
                           preferred_element_type=jnp.float32)


def _mm_nt(a, b):
    return lax.dot_general(a, b, (((1,), (1,)), ((), ())),
                           preferred_element_type=jnp.float32)


def _dense_body(xn_ref, nsum_ref, tsum_ref, idc_ref, bd_ref,
                wenc_ref, benc_ref, wq_ref, wo_ref,
                wc_ref, bc_ref, x_ref, emb_ref, lb_ref):
    f32 = jnp.float32
    bf16 = jnp.bfloat16
    xn = xn_ref[...]
    agg = (nsum_ref[...] + tsum_ref[...]) * (1.0 / (NBR_ + 1e-9))

    aggx = jnp.broadcast_to(agg[:, None, :], (2 * BB2, E_, D_MODEL))
    x_in = xn + aggx.reshape(BBT, D_MODEL)
    xe = jnp.tanh(_mm(x_in.astype(bf16), wenc_ref[...])
                  + benc_ref[...])
    xr = xe.reshape(BB2, 2, E_, D_MODEL)
    x_ref[0] = xr[:, 0]
    x_ref[1] = xr[:, 1]

    mcol = (idc_ref[...] != 0).astype(f32)
    mcolb = (idc_ref[...] != 0).astype(bf16)
    bd = bd_ref[...]

    xeb = xe.astype(bf16)
    qkv = _mm(xeb, wq_ref[...]).astype(bf16)
    q = lax.slice(qkv, (0, 0), (BBT, D_MODEL))
    k = lax.slice(qkv, (0, D_MODEL), (BBT, 2 * D_MODEL))
    v = lax.slice(qkv, (0, 2 * D_MODEL), (BBT, 3 * D_MODEL))
    ohs = []
    for h in range(N_HEAD):
        qh = lax.slice(q, (0, h * D_K), (BBT, (h + 1) * D_K))
        kh = lax.slice(k, (0, h * D_K), (BBT, (h + 1) * D_K))
        vh = lax.slice(v, (0, h * D_V), (BBT, (h + 1) * D_V))
        sh = _mm_nt(qh, kh).astype(bf16)
        p = jnp.exp(jnp.minimum(sh, bf16(80.0))) * bd * mcolb
        dn = jnp.sum(p, axis=1, keepdims=True).astype(f32) + 1e-30
        ohs.append((_mm(p, vh) / dn).astype(bf16))
    outacc = _mm(jnp.concatenate(ohs, axis=1), wo_ref[...])

    ri2 = lax.broadcasted_iota(jnp.int32, (BB2, BBT), 0)
    ci2 = lax.broadcasted_iota(jnp.int32, (BB2, BBT), 1)
    sel = ((ci2 // T_) == ri2).astype(f32) * mcol
    cnt = jnp.sum(sel, axis=1, keepdims=True)
    emb = _mm(sel, outacc) / (cnt + 1e-9)
    emb_ref[...] = emb
    lb_ref[...] = jax.nn.sigmoid(_mm(emb, wc_ref[...]) + bc_ref[...])


def _dense(xn, nsum, tsum, idc, bd, wenc, benc, wqkv, wo4, wc, bc2):
    return pl.pallas_call(
        _dense_body,
        grid=(B_ // BB2,),
        in_specs=[
            pl.BlockSpec((BBT, D_MODEL), lambda i: (i, 0)),
            pl.BlockSpec((2 * BB2, D_MODEL), lambda i: (i, 0)),
            pl.BlockSpec((2 * BB2, D_MODEL), lambda i: (i, 0)),
            pl.BlockSpec((1, BBT), lambda i: (0, i)),
            pl.BlockSpec((BBT, BBT), lambda i: (0, 0)),
            pl.BlockSpec((D_MODEL, D_MODEL), lambda i: (0, 0)),
            pl.BlockSpec((1, D_MODEL), lambda i: (0, 0)),
            pl.BlockSpec((D_MODEL, 3 * D_MODEL), lambda i: (0, 0)),
            pl.BlockSpec((D_MODEL, D_MODEL), lambda i: (0, 0)),
            pl.BlockSpec((D_MODEL, 1), lambda i: (0, 0)),
            pl.BlockSpec((1, 1), lambda i: (0, 0)),
        ],
        out_specs=[
            pl.BlockSpec((2, BB2, E_, D_MODEL), lambda i: (0, i, 0, 0)),
            pl.BlockSpec((BB2, D_MODEL), lambda i: (i, 0)),
            pl.BlockSpec((BB2, 1), lambda i: (i, 0)),
        ],
        out_shape=[
            jax.ShapeDtypeStruct((2, B_, E_, D_MODEL), jnp.float32),
            jax.ShapeDtypeStruct((B_, D_MODEL), jnp.float32),
            jax.ShapeDtypeStruct((B_, 1), jnp.float32),
        ],
        interpret=_INTERPRET,
    )(xn, nsum, tsum, idc, bd, wenc, benc, wqkv, wo4, wc, bc2)


def kernel(memory, batch_hyperedge, batch_h_index, cur_time,
           batch_h_index_times, batch_h_index_mask, W_enc, b_enc, w_time,
           b_time, Wq, Wk, Wv, Wo, Wc, bc):
    bh = batch_hyperedge.astype(jnp.int32)
    ids_tok = jnp.concatenate([bh[0], bh[1]], axis=1)
    ids_flat = ids_tok.reshape(B_ * T_)
    nbr_ids = batch_h_index.astype(jnp.int32).reshape(B_ * 2 * NBR_)

    xn, nsum = _sc_gather(memory, ids_flat, nbr_ids)

    dtrow = (cur_time[:, :, None] - batch_h_index_times).reshape(1, B_ * 2 * NBR_)

    bf16 = jnp.bfloat16
    seg_mat = jnp.asarray(
        np.kron(np.eye(SEGBLK, dtype=np.float32),
                np.ones((NBR_, 1), np.float32)),
        dtype=bf16)
    tsum = _tfeat_sum(dtrow, w_time.reshape(D_MODEL, 1),
                      b_time.reshape(D_MODEL, 1), seg_mat).T

    wqkv = jnp.concatenate(
        [Wq * (1.0 / np.sqrt(D_K)), Wk, Wv], axis=1).astype(bf16)
    wo4 = Wo.astype(bf16)

    bd = jnp.asarray(
        np.kron(np.eye(BB2, dtype=np.float32),
                np.ones((T_, T_), np.float32)), dtype=bf16)
    x4, emb, lb = _dense(
        xn, nsum, tsum,
        ids_flat.reshape(1, B_ * T_), bd,
        W_enc.astype(bf16), b_enc.reshape(1, D_MODEL), wqkv, wo4,
        Wc, bc.reshape(1, 1))
    return lb, emb, x4
